# Initial kernel scaffold; baseline (speedup 1.0000x reference)
import jax
import jax.numpy as jnp
from jax.experimental import pallas as pl


def kernel(x, edge_index, edge_attr, u, batch, W1a, b1a, W1b, b1b, W2a, b2a, W2b, b2b):
    raise NotImplementedError("write your pallas kernel here")


# speedup vs baseline: 1.6965x; 1.6965x over previous
import functools

import jax
import jax.numpy as jnp
from jax import lax
from jax.experimental import pallas as pl
from jax.experimental.pallas import tpu as pltpu
from jax.experimental.pallas import tpu_sc as plsc

_N = 10000
_E = 320000
_DN = 128
_DE = 16
_H = 128

_CH = 128
_NCHUNK = _E // _CH
_PW = 80
_HF = 64
_NC = 2
_NS = 16
_NPAD = 10240
_RPT = _NPAD // _NS
_ZR = 128


def _preh_body(x_ref, wxa_ref, wxb_ref, ha_ref, hb_ref):
    xb = x_ref[...]
    ha_ref[...] = jnp.dot(xb, wxa_ref[...], preferred_element_type=jnp.float32)
    hb_ref[...] = jnp.dot(xb, wxb_ref[...], preferred_element_type=jnp.float32)


def _preh_call(x, wxa, wxb):
    blk = 1000
    return pl.pallas_call(
        _preh_body,
        grid=(_N // blk,),
        in_specs=[pl.BlockSpec((blk, _DN), lambda i: (i, 0)),
                  pl.BlockSpec((_DN, _PW), lambda i: (0, 0)),
                  pl.BlockSpec((_DN, _PW), lambda i: (0, 0))],
        out_specs=[pl.BlockSpec((blk, _PW), lambda i: (i, 0)),
                   pl.BlockSpec((blk, _PW), lambda i: (i, 0))],
        out_shape=[jax.ShapeDtypeStruct((_N, _PW), jnp.float32),
                   jax.ShapeDtypeStruct((_N, _PW), jnp.float32)],
    )(x, wxa, wxb)


def _pree_body(e_ref, wea_ref, web_ref, ba_ref, bb_ref, ea_ref, eb_ref):
    eb = e_ref[...]
    ea_ref[...] = jnp.dot(eb, wea_ref[...],
                          preferred_element_type=jnp.float32) + ba_ref[...]
    eb_ref[...] = jnp.dot(eb, web_ref[...],
                          preferred_element_type=jnp.float32) + bb_ref[...]


def _pree_call(e, wea, web, ba, bb):
    blk = 8000
    return pl.pallas_call(
        _pree_body,
        grid=(_E // blk,),
        in_specs=[pl.BlockSpec((blk, _DE), lambda i: (i, 0)),
                  pl.BlockSpec((_DE, _PW), lambda i: (0, 0)),
                  pl.BlockSpec((_DE, _PW), lambda i: (0, 0)),
                  pl.BlockSpec((1, _PW), lambda i: (0, 0)),
                  pl.BlockSpec((1, _PW), lambda i: (0, 0))],
        out_specs=[pl.BlockSpec((blk, _PW), lambda i: (i, 0)),
                   pl.BlockSpec((blk, _PW), lambda i: (i, 0))],
        out_shape=[jax.ShapeDtypeStruct((_E, _PW), jnp.float32),
                   jax.ShapeDtypeStruct((_E, _PW), jnp.float32)],
    )(e, wea, web, ba, bb)


def _epi_body(p_ref, x_ref, w1b_ref, b1b_ref, w2a_ref, b2a_ref, w2b_ref,
              b2b_ref, o_ref):
    sa = p_ref[0]
    sb = p_ref[1]
    cnt = sa[:, _HF:_HF + 1]
    ssum = jnp.concatenate([sa[:, :_HF], sb[:, :_HF]], axis=1)
    mean = ssum / jnp.maximum(cnt, 1.0)
    agg = (jnp.dot(mean, w1b_ref[...], preferred_element_type=jnp.float32)
           + b1b_ref[...] * (cnt > 0.0).astype(jnp.float32))
    xb = x_ref[...]
    h = jnp.maximum(
        jnp.dot(xb, w2a_ref[:_DN], preferred_element_type=jnp.float32)
        + jnp.dot(agg, w2a_ref[_DN:], preferred_element_type=jnp.float32)
        + b2a_ref[...], 0.0)
    o_ref[...] = (jnp.dot(h, w2b_ref[...], preferred_element_type=jnp.float32)
                  + b2b_ref[...] + xb)


def _epi_call(part, x, w1b, b1b, w2a, b2a, w2b, b2b):
    blk = 1000
    return pl.pallas_call(
        _epi_body,
        grid=(_N // blk,),
        in_specs=[pl.BlockSpec((2, blk, _PW), lambda i: (0, i, 0)),
                  pl.BlockSpec((blk, _DN), lambda i: (i, 0)),
                  pl.BlockSpec((_H, _H), lambda i: (0, 0)),
                  pl.BlockSpec((1, _H), lambda i: (0, 0)),
                  pl.BlockSpec((_H + _DN, _H), lambda i: (0, 0)),
                  pl.BlockSpec((1, _H), lambda i: (0, 0)),
                  pl.BlockSpec((_H, _DN), lambda i: (0, 0)),
                  pl.BlockSpec((1, _DN), lambda i: (0, 0))],
        out_specs=pl.BlockSpec((blk, _DN), lambda i: (i, 0)),
        out_shape=jax.ShapeDtypeStruct((_N, _DN), jnp.float32),
    )(part, x, w1b, b1b, w2a, b2a, w2b, b2b)


_MESH = plsc.VectorSubcoreMesh(core_axis_name="c", subcore_axis_name="s")


@functools.partial(
    pl.kernel,
    out_type=jax.ShapeDtypeStruct((_NC, _NPAD, _PW), jnp.float32),
    mesh=_MESH,
    scratch_types=[
        pltpu.VMEM((_CH,), jnp.int32),
        pltpu.VMEM((_CH,), jnp.int32),
        pltpu.VMEM((_CH, _PW), jnp.float32),
        pltpu.VMEM((_CH, _PW), jnp.float32),
        pltpu.VMEM((_ZR, _PW), jnp.float32),
        pltpu.VMEM_SHARED((_NPAD, _PW), jnp.float32),
        pltpu.SemaphoreType.DMA,
    ],
    compiler_params=pltpu.CompilerParams(use_tc_tiling_on_sc=False))
def _sc_edge(h1a_hbm, h1b_hbm, ewa_hbm, ewb_hbm, row_hbm, col_hbm, out_hbm,
             colv, rowv, ebuf, hbuf, zbuf, seg, sem):
    cid = lax.axis_index("c")
    sid = lax.axis_index("s")

    def zrow(r, carry):
        for g in range(_PW // 16):
            zbuf[r, pl.ds(g * 16, 16)] = jnp.zeros((16,), jnp.float32)
        return carry

    lax.fori_loop(0, _ZR, zrow, 0)
    for k in range(_RPT // _ZR):
        pltpu.sync_copy(zbuf, seg.at[pl.ds(sid * _RPT + k * _ZR, _ZR)])
    plsc.subcore_barrier()

    tmax = (_NCHUNK + _NS - 1) // _NS

    def run_half(h1_hbm, ew_hbm):
        def chunk_body(t, carry):
            chunk = sid + t * _NS

            @pl.when(chunk < _NCHUNK)
            def _():
                off = chunk * _CH
                pltpu.sync_copy(col_hbm.at[pl.ds(off, _CH)], colv)
                pltpu.sync_copy(row_hbm.at[pl.ds(off, _CH)], rowv)
                pltpu.sync_copy(ew_hbm.at[pl.ds(off, _CH)], ebuf)
                pltpu.async_copy(h1_hbm.at[colv], hbuf, sem).wait()

                def rbody(r, c2):
                    for g in range(_PW // 16):
                        sl = pl.ds(g * 16, 16)
                        ebuf[r, sl] = jnp.maximum(ebuf[r, sl] + hbuf[r, sl],
                                                  0.0)
                    return c2

                lax.fori_loop(0, _CH, rbody, 0)
                pltpu.sync_copy(ebuf, seg.at[rowv], add=True)

            return carry

        lax.fori_loop(0, tmax, chunk_body, 0)

    @pl.when(cid == 0)
    def _():
        run_half(h1a_hbm, ewa_hbm)

    @pl.when(cid == 1)
    def _():
        run_half(h1b_hbm, ewb_hbm)

    plsc.subcore_barrier()
    pltpu.sync_copy(seg.at[pl.ds(sid * _RPT, _RPT)],
                    out_hbm.at[cid, pl.ds(sid * _RPT, _RPT)])


def kernel(x, edge_index, edge_attr, u, batch,
           W1a, b1a, W1b, b1b, W2a, b2a, W2b, b2b):
    del u, batch
    row = edge_index[0]
    col = edge_index[1]
    wxa = jnp.zeros((_DN, _PW), jnp.float32).at[:, :_HF].set(W1a[:_DN, :_HF])
    wxb = jnp.zeros((_DN, _PW), jnp.float32).at[:, :_HF].set(W1a[:_DN, _HF:])
    wea = jnp.zeros((_DE, _PW), jnp.float32).at[:, :_HF].set(W1a[_DN:, :_HF])
    web = jnp.zeros((_DE, _PW), jnp.float32).at[:, :_HF].set(W1a[_DN:, _HF:])
    ba = (jnp.zeros((1, _PW), jnp.float32)
          .at[0, :_HF].set(b1a[:_HF]).at[0, _HF].set(1.0))
    bb = jnp.zeros((1, _PW), jnp.float32).at[0, :_HF].set(b1a[_HF:])

    h1a, h1b = _preh_call(x, wxa, wxb)
    ewa, ewb = _pree_call(edge_attr, wea, web, ba, bb)
    part = _sc_edge(h1a, h1b, ewa, ewb, row, col)
    return _epi_call(part, x, W1b, b1b.reshape(1, _H), W2a,
                     b2a.reshape(1, _H), W2b, b2b.reshape(1, _DN))
